# baseline (speedup 1.0000x reference)
import functools

import jax
import jax.numpy as jnp
from jax import lax
from jax.experimental import pallas as pl
from jax.experimental.pallas import tpu as pltpu
from jax.experimental.pallas import tpu_sc as plsc

N = 10000
E = 320000
D = 128
H = 64
B = 64

NC = 2
NS = 16
NW = NC * NS
C = 128
NCHUNK = 80
E_PAD = NW * NCHUNK * C
TOT_CHUNKS = E_PAD // C
K0 = NCHUNK
K1 = 2 * NCHUNK - K0
N_PAD = 10240
RPT = N_PAD // NS


def _sc_deg_body(dst2, ones_hbm, zero16_hbm, out, didx, onesv, acc, sem):
    cid = lax.axis_index("c")
    sid = lax.axis_index("s")
    wid = cid * NS + sid
    lo = sid * RPT
    pltpu.sync_copy(zero16_hbm, acc.at[pl.ds(lo, RPT)])
    pltpu.sync_copy(ones_hbm, onesv)
    pltpu.sync_copy(dst2.at[pl.ds(wid * NCHUNK, NCHUNK)], didx)
    plsc.subcore_barrier()

    def body(bi, carry):
        descs = [
            pltpu.async_copy(onesv, acc.at[didx.at[8 * bi + i]], sem, add=True)
            for i in range(8)
        ]
        for d in descs:
            d.wait()
        return carry

    lax.fori_loop(0, NCHUNK // 8, body, 0)
    plsc.subcore_barrier()
    pltpu.sync_copy(acc.at[pl.ds(lo, RPT)], out.at[cid].at[pl.ds(lo, RPT)])


def _sc_gs_body(g, src2, dst2, zrows_hbm, out,
                sidx, didx, r0, r1, r2, r3,
                acc, g0, g1, g2, g3, s0, s1, s2, s3):
    cid = lax.axis_index("c")
    sid = lax.axis_index("s")
    lo = sid * RPT
    rows = (r0, r1, r2, r3)
    gsem = (g0, g1, g2, g3)
    ssem = (s0, s1, s2, s3)
    nch = jnp.where(cid == 0, K0, K1)
    with jax.named_scope("gs_init"):
        pltpu.sync_copy(zrows_hbm, acc.at[pl.ds(lo, RPT)])

        @pl.when(cid == 0)
        def _():
            pltpu.sync_copy(src2.at[pl.ds(sid * K0, K0)],
                            sidx.at[pl.ds(0, K0)])
            pltpu.sync_copy(dst2.at[pl.ds(sid * K0, K0)],
                            didx.at[pl.ds(0, K0)])

        @pl.when(cid != 0)
        def _():
            base = NS * K0 + sid * K1
            pltpu.sync_copy(src2.at[pl.ds(base, K1)],
                            sidx.at[pl.ds(0, K1)])
            pltpu.sync_copy(dst2.at[pl.ds(base, K1)],
                            didx.at[pl.ds(0, K1)])

        plsc.subcore_barrier()

    def start_gather(i, chunk):
        pltpu.async_copy(g.at[sidx.at[chunk]], rows[i], gsem[i])

    def wait_gather(i):
        pltpu.make_async_copy(g.at[sidx.at[0]], rows[i], gsem[i]).wait()

    def start_scatter(i, chunk):
        pltpu.async_copy(rows[i], acc.at[didx.at[chunk]], ssem[i], add=True)

    def wait_scatter(i):
        pltpu.make_async_copy(rows[i], acc.at[didx.at[0]], ssem[i]).wait()

    with jax.named_scope("gs_loop"):
        for i in range(4):
            start_gather(i, i)

        def body(j, carry):
            base = 4 * j
            for i in range(4):
                wait_gather(i)
                start_scatter(i, base + i)
            for i in range(4):
                wait_scatter(i)
                start_gather(i, base + 4 + i)
            return carry

        lax.fori_loop(0, nch // 4 - 1, body, 0)
        base = nch - 4
        for i in range(4):
            wait_gather(i)
            start_scatter(i, base + i)
        for i in range(4):
            wait_scatter(i)
    with jax.named_scope("gs_wb"):
        plsc.subcore_barrier()
        pltpu.sync_copy(acc.at[pl.ds(lo, RPT)],
                        out.at[cid].at[pl.ds(lo, RPT)])


@functools.lru_cache(maxsize=None)
def _build_sc():
    mesh = plsc.VectorSubcoreMesh(core_axis_name="c", subcore_axis_name="s")
    params = pltpu.CompilerParams(use_tc_tiling_on_sc=False)
    sc_deg = pl.kernel(
        _sc_deg_body,
        out_type=jax.ShapeDtypeStruct((NC, N_PAD, 16), jnp.float32),
        mesh=mesh,
        compiler_params=params,
        scratch_types=[
            pltpu.VMEM((NCHUNK, C), jnp.int32),
            pltpu.VMEM((C, 16), jnp.float32),
            pltpu.VMEM_SHARED((N_PAD, 16), jnp.float32),
            pltpu.SemaphoreType.DMA,
        ],
    )
    sc_gs = pl.kernel(
        _sc_gs_body,
        out_type=jax.ShapeDtypeStruct((NC, N_PAD, H), jnp.float32),
        mesh=mesh,
        compiler_params=params,
        scratch_types=[
            pltpu.VMEM((K0, C), jnp.int32),
            pltpu.VMEM((K0, C), jnp.int32),
            pltpu.VMEM((C, H), jnp.float32),
            pltpu.VMEM((C, H), jnp.float32),
            pltpu.VMEM((C, H), jnp.float32),
            pltpu.VMEM((C, H), jnp.float32),
            pltpu.VMEM_SHARED((N_PAD, H), jnp.float32),
        ] + [pltpu.SemaphoreType.DMA] * 8,
    )
    return sc_deg, sc_gs


def _tc_prep_body(degp_ref, x_ref, w_ref, dinv_ref, g_ref):
    degp = degp_ref[...]
    deg = degp[0, :, 0:1] + degp[1, :, 0:1] + 1.0
    dinv = lax.rsqrt(jnp.maximum(deg, 1.0))
    h = jnp.dot(x_ref[...], w_ref[...], preferred_element_type=jnp.float32)
    dinv_ref[...] = dinv
    g_ref[...] = h * dinv


def _tc_mid_body(p_ref, g_ref, dinv_ref, b_ref, w_ref, gout_ref):
    p = p_ref[...]
    dinv = dinv_ref[...]
    accum = p[0] + p[1] + g_ref[...]
    xn = jnp.maximum(accum * dinv + b_ref[...], 0.0)
    gout_ref[...] = (
        jnp.dot(xn, w_ref[...], preferred_element_type=jnp.float32) * dinv
    )


def _tc_final_body(p_ref, g_ref, dinv_ref, b_ref, batch_ref, out_ref):
    p = p_ref[...]
    accum = p[0] + p[1] + g_ref[...]
    h = jnp.maximum(accum * dinv_ref[...] + b_ref[...], 0.0)
    seg = batch_ref[...]
    ids = lax.broadcasted_iota(jnp.int32, (B, N_PAD), 0)
    oh = (ids == seg).astype(jnp.float32)
    sums = jnp.dot(oh, h, preferred_element_type=jnp.float32)
    cnt = jnp.sum(oh, axis=1, keepdims=True)
    out_ref[...] = sums / jnp.maximum(cnt, 1.0)


_tc_prep = pl.pallas_call(
    _tc_prep_body,
    out_shape=[
        jax.ShapeDtypeStruct((N_PAD, 1), jnp.float32),
        jax.ShapeDtypeStruct((N_PAD, H), jnp.float32),
    ],
)

_tc_mid = pl.pallas_call(
    _tc_mid_body,
    out_shape=jax.ShapeDtypeStruct((N_PAD, H), jnp.float32),
)

_tc_final = pl.pallas_call(
    _tc_final_body,
    out_shape=jax.ShapeDtypeStruct((B, H), jnp.float32),
)


def kernel(x, edge_index, batch, W1, b1, W2, b2, W3, b3, W4, b4):
    src = edge_index[0]
    dst = edge_index[1]
    pad = E_PAD - E
    pad_dst = N + (jnp.arange(pad, dtype=jnp.int32) % (N_PAD - N))
    src2 = jnp.concatenate(
        [src, jnp.zeros((pad,), jnp.int32)]).reshape(TOT_CHUNKS, C)
    dst2 = jnp.concatenate([dst, pad_dst]).reshape(TOT_CHUNKS, C)
    xp = jnp.pad(x, ((0, N_PAD - N), (0, 0)))
    batch_row = jnp.pad(batch, (0, N_PAD - N),
                        constant_values=B).reshape(1, N_PAD)
    ones16 = jnp.ones((C, 16), jnp.float32)
    zeros16 = jnp.zeros((RPT, 16), jnp.float32)
    zrows = jnp.zeros((RPT, H), jnp.float32)
    b1r, b2r, b3r, b4r = (b.reshape(1, H) for b in (b1, b2, b3, b4))

    sc_deg, sc_gs = _build_sc()
    degp = sc_deg(dst2, ones16, zeros16)
    dinv, g = _tc_prep(degp, xp, W1)
    p = sc_gs(g, src2, dst2, zrows)
    g = _tc_mid(p, g, dinv, b1r, W2)
    p = sc_gs(g, src2, dst2, zrows)
    g = _tc_mid(p, g, dinv, b2r, W3)
    p = sc_gs(g, src2, dst2, zrows)
    g = _tc_mid(p, g, dinv, b3r, W4)
    p = sc_gs(g, src2, dst2, zrows)
    return _tc_final(p, g, dinv, b4r, batch_row)


# speedup vs baseline: 2.3526x; 2.3526x over previous
import functools

import jax
import jax.numpy as jnp
from jax import lax
from jax.experimental import pallas as pl
from jax.experimental.pallas import tpu as pltpu
from jax.experimental.pallas import tpu_sc as plsc

N = 10000
E = 320000
D = 128
H = 64
B = 64

NC = 2
NS = 16
NW = NC * NS
C = 128
NCHUNK = 80
E_PAD = NW * NCHUNK * C
TOT_CHUNKS = E_PAD // C
K0 = NCHUNK
K1 = 2 * NCHUNK - K0
N_PAD = 10240
RPT = N_PAD // NS


def _sc_deg_body(dst2, ones_hbm, zero16_hbm, out, didx, onesv, acc, sem):
    cid = lax.axis_index("c")
    sid = lax.axis_index("s")
    wid = cid * NS + sid
    lo = sid * RPT
    pltpu.sync_copy(zero16_hbm, acc.at[pl.ds(lo, RPT)])
    pltpu.sync_copy(ones_hbm, onesv)
    pltpu.sync_copy(dst2.at[pl.ds(wid * NCHUNK, NCHUNK)], didx)
    plsc.subcore_barrier()

    def body(bi, carry):
        descs = [
            pltpu.async_copy(onesv, acc.at[didx.at[8 * bi + i]], sem, add=True)
            for i in range(8)
        ]
        for d in descs:
            d.wait()
        return carry

    lax.fori_loop(0, NCHUNK // 8, body, 0)
    plsc.subcore_barrier()
    pltpu.sync_copy(acc.at[pl.ds(lo, RPT)], out.at[cid].at[pl.ds(lo, RPT)])


def _sc_gs_body(g, src2, dst2, zrows_hbm, out,
                sidx, didx, r0, r1, r2, r3,
                acc, g0, g1, g2, g3, s0, s1, s2, s3):
    cid = lax.axis_index("c")
    sid = lax.axis_index("s")
    lo = sid * RPT
    rows = (r0, r1, r2, r3)
    gsem = (g0, g1, g2, g3)
    ssem = (s0, s1, s2, s3)
    nch = jnp.where(cid == 0, K0, K1)
    with jax.named_scope("gs_init"):
        pltpu.sync_copy(zrows_hbm, acc.at[pl.ds(lo, RPT)])

        @pl.when(cid == 0)
        def _():
            pltpu.sync_copy(src2.at[pl.ds(sid * K0, K0)],
                            sidx.at[pl.ds(0, K0)])
            pltpu.sync_copy(dst2.at[pl.ds(sid * K0, K0)],
                            didx.at[pl.ds(0, K0)])

        @pl.when(cid != 0)
        def _():
            base = NS * K0 + sid * K1
            pltpu.sync_copy(src2.at[pl.ds(base, K1)],
                            sidx.at[pl.ds(0, K1)])
            pltpu.sync_copy(dst2.at[pl.ds(base, K1)],
                            didx.at[pl.ds(0, K1)])

        plsc.subcore_barrier()

    def start_gather(i, chunk):
        pltpu.async_copy(g.at[sidx.at[chunk]], rows[i], gsem[i])

    def wait_gather(i):
        pltpu.make_async_copy(g.at[sidx.at[0]], rows[i], gsem[i]).wait()

    def start_scatter(i, chunk):
        pltpu.async_copy(rows[i], acc.at[didx.at[chunk]], ssem[i], add=True)

    def wait_scatter(i):
        pltpu.make_async_copy(rows[i], acc.at[didx.at[0]], ssem[i]).wait()

    with jax.named_scope("gs_loop"):
        for i in range(4):
            start_gather(i, i)

        def body(j, carry):
            base = 4 * j
            for i in range(4):
                wait_gather(i)
                start_scatter(i, base + i)
            for i in range(4):
                wait_scatter(i)
                start_gather(i, base + 4 + i)
            return carry

        lax.fori_loop(0, nch // 4 - 1, body, 0)
        base = nch - 4
        for i in range(4):
            wait_gather(i)
            start_scatter(i, base + i)
        for i in range(4):
            wait_scatter(i)
    with jax.named_scope("gs_wb"):
        plsc.subcore_barrier()
        pltpu.sync_copy(acc.at[pl.ds(lo, RPT)],
                        out.at[cid].at[pl.ds(lo, RPT)])


@functools.lru_cache(maxsize=None)
def _build_sc():
    mesh = plsc.VectorSubcoreMesh(core_axis_name="c", subcore_axis_name="s")
    params = pltpu.CompilerParams(use_tc_tiling_on_sc=False)
    sc_deg = pl.kernel(
        _sc_deg_body,
        out_type=jax.ShapeDtypeStruct((NC, N_PAD, 16), jnp.float32),
        mesh=mesh,
        compiler_params=params,
        scratch_types=[
            pltpu.VMEM((NCHUNK, C), jnp.int32),
            pltpu.VMEM((C, 16), jnp.float32),
            pltpu.VMEM_SHARED((N_PAD, 16), jnp.float32),
            pltpu.SemaphoreType.DMA,
        ],
    )
    sc_gs = pl.kernel(
        _sc_gs_body,
        out_type=jax.ShapeDtypeStruct((NC, N_PAD, H), jnp.float32),
        mesh=mesh,
        compiler_params=params,
        scratch_types=[
            pltpu.VMEM((K0, C), jnp.int32),
            pltpu.VMEM((K0, C), jnp.int32),
            pltpu.VMEM((C, H), jnp.float32),
            pltpu.VMEM((C, H), jnp.float32),
            pltpu.VMEM((C, H), jnp.float32),
            pltpu.VMEM((C, H), jnp.float32),
            pltpu.VMEM_SHARED((N_PAD, H), jnp.float32),
        ] + [pltpu.SemaphoreType.DMA] * 8,
    )
    return sc_deg, sc_gs


def _tc_prep_body(degp_ref, x_ref, w_ref, dinv_ref, g_ref):
    degp = degp_ref[...]
    deg = degp[0, :, 0:1] + degp[1, :, 0:1] + 1.0
    dinv = lax.rsqrt(jnp.maximum(deg, 1.0))
    h = jnp.dot(x_ref[...], w_ref[...], preferred_element_type=jnp.float32)
    dinv_ref[...] = dinv
    g_ref[...] = h * dinv


def _tc_mid_body(p_ref, g_ref, dinv_ref, b_ref, w_ref, gout_ref):
    p = p_ref[...]
    dinv = dinv_ref[...]
    accum = p[0] + p[1] + g_ref[...]
    xn = jnp.maximum(accum * dinv + b_ref[...], 0.0)
    gout_ref[...] = (
        jnp.dot(xn, w_ref[...], preferred_element_type=jnp.float32) * dinv
    )


def _tc_final_body(p_ref, g_ref, dinv_ref, b_ref, batch_ref, out_ref):
    p = p_ref[...]
    accum = p[0] + p[1] + g_ref[...]
    h = jnp.maximum(accum * dinv_ref[...] + b_ref[...], 0.0)
    seg = batch_ref[...]
    ids = lax.broadcasted_iota(jnp.int32, (B, N_PAD), 0)
    oh = (ids == seg).astype(jnp.float32)
    sums = jnp.dot(oh, h, preferred_element_type=jnp.float32)
    cnt = jnp.sum(oh, axis=1, keepdims=True)
    out_ref[...] = sums / jnp.maximum(cnt, 1.0)


_tc_prep = pl.pallas_call(
    _tc_prep_body,
    out_shape=[
        jax.ShapeDtypeStruct((N_PAD, 1), jnp.float32),
        jax.ShapeDtypeStruct((N_PAD, H), jnp.float32),
    ],
)

_tc_mid = pl.pallas_call(
    _tc_mid_body,
    out_shape=jax.ShapeDtypeStruct((N_PAD, H), jnp.float32),
)

_tc_final = pl.pallas_call(
    _tc_final_body,
    out_shape=jax.ShapeDtypeStruct((B, H), jnp.float32),
)


def kernel(x, edge_index, batch, W1, b1, W2, b2, W3, b3, W4, b4):
    src = edge_index[0]
    dst = edge_index[1]
    ppw = (E_PAD - E) // NW
    rpw = E // NW
    pad_src = jnp.broadcast_to(jnp.arange(ppw, dtype=jnp.int32), (NW, ppw))
    pad_dst = pad_src + N
    src2 = jnp.concatenate(
        [src.reshape(NW, rpw), pad_src], axis=1).reshape(TOT_CHUNKS, C)
    dst2 = jnp.concatenate(
        [dst.reshape(NW, rpw), pad_dst], axis=1).reshape(TOT_CHUNKS, C)
    xp = jnp.pad(x, ((0, N_PAD - N), (0, 0)))
    batch_row = jnp.pad(batch, (0, N_PAD - N),
                        constant_values=B).reshape(1, N_PAD)
    ones16 = jnp.ones((C, 16), jnp.float32)
    zeros16 = jnp.zeros((RPT, 16), jnp.float32)
    zrows = jnp.zeros((RPT, H), jnp.float32)
    b1r, b2r, b3r, b4r = (b.reshape(1, H) for b in (b1, b2, b3, b4))

    sc_deg, sc_gs = _build_sc()
    degp = sc_deg(dst2, ones16, zeros16)
    dinv, g = _tc_prep(degp, xp, W1)
    p = sc_gs(g, src2, dst2, zrows)
    g = _tc_mid(p, g, dinv, b1r, W2)
    p = sc_gs(g, src2, dst2, zrows)
    g = _tc_mid(p, g, dinv, b2r, W3)
    p = sc_gs(g, src2, dst2, zrows)
    g = _tc_mid(p, g, dinv, b3r, W4)
    p = sc_gs(g, src2, dst2, zrows)
    return _tc_final(p, g, dinv, b4r, batch_row)


# speedup vs baseline: 2.3857x; 1.0141x over previous
import functools

import jax
import jax.numpy as jnp
from jax import lax
from jax.experimental import pallas as pl
from jax.experimental.pallas import tpu as pltpu
from jax.experimental.pallas import tpu_sc as plsc

N = 10000
E = 320000
D = 128
H = 64
B = 64

NC = 2
NS = 16
NW = NC * NS
C = 128
NCHUNK = 80
E_PAD = NW * NCHUNK * C
TOT_CHUNKS = E_PAD // C
K0 = NCHUNK
K1 = 2 * NCHUNK - K0
N_PAD = 10240
RPT = N_PAD // NS


def _sc_deg_body(dst2, ones_hbm, zero16_hbm, out, didx, onesv, acc, sem):
    cid = lax.axis_index("c")
    sid = lax.axis_index("s")
    wid = cid * NS + sid
    lo = sid * RPT
    pltpu.sync_copy(zero16_hbm, acc.at[pl.ds(lo, RPT)])
    pltpu.sync_copy(ones_hbm, onesv)
    pltpu.sync_copy(dst2.at[pl.ds(wid * NCHUNK, NCHUNK)], didx)
    plsc.subcore_barrier()

    def body(bi, carry):
        descs = [
            pltpu.async_copy(onesv, acc.at[didx.at[8 * bi + i]], sem, add=True)
            for i in range(8)
        ]
        for d in descs:
            d.wait()
        return carry

    lax.fori_loop(0, NCHUNK // 8, body, 0)
    plsc.subcore_barrier()
    pltpu.sync_copy(acc.at[pl.ds(lo, RPT)], out.at[cid].at[pl.ds(lo, RPT)])


def _sc_gs_body(g, src2, dst2, zrows_hbm, out,
                sidx, didx, r0, r1, r2, r3,
                acc, g0, g1, g2, g3, s0, s1, s2, s3):
    cid = lax.axis_index("c")
    sid = lax.axis_index("s")
    lo = sid * RPT
    rows = (r0, r1, r2, r3)
    gsem = (g0, g1, g2, g3)
    ssem = (s0, s1, s2, s3)
    nch = jnp.where(cid == 0, K0, K1)
    with jax.named_scope("gs_init"):
        pltpu.async_copy(zrows_hbm, acc.at[pl.ds(lo, RPT)], s0)

        @pl.when(cid == 0)
        def _():
            pltpu.async_copy(src2.at[pl.ds(sid * K0, K0)],
                             sidx.at[pl.ds(0, K0)], s1)
            pltpu.async_copy(dst2.at[pl.ds(sid * K0, K0)],
                             didx.at[pl.ds(0, K0)], s2)

        @pl.when(cid != 0)
        def _():
            base = NS * K0 + sid * K1
            pltpu.async_copy(src2.at[pl.ds(base, K1)],
                             sidx.at[pl.ds(0, K1)], s1)
            pltpu.async_copy(dst2.at[pl.ds(base, K1)],
                             didx.at[pl.ds(0, K1)], s2)

        pltpu.make_async_copy(zrows_hbm, acc.at[pl.ds(lo, RPT)], s0).wait()
        pltpu.make_async_copy(src2.at[pl.ds(0, K0)],
                              sidx.at[pl.ds(0, K0)], s1).wait()
        pltpu.make_async_copy(dst2.at[pl.ds(0, K0)],
                              didx.at[pl.ds(0, K0)], s2).wait()
        plsc.subcore_barrier()

    def start_gather(i, chunk):
        pltpu.async_copy(g.at[sidx.at[chunk]], rows[i], gsem[i])

    def wait_gather(i):
        pltpu.make_async_copy(g.at[sidx.at[0]], rows[i], gsem[i]).wait()

    def start_scatter(i, chunk):
        pltpu.async_copy(rows[i], acc.at[didx.at[chunk]], ssem[i], add=True)

    def wait_scatter(i):
        pltpu.make_async_copy(rows[i], acc.at[didx.at[0]], ssem[i]).wait()

    with jax.named_scope("gs_loop"):
        for i in range(4):
            start_gather(i, i)

        def body(j, carry):
            base = 4 * j
            for i in range(4):
                wait_gather(i)
                start_scatter(i, base + i)
            for i in range(4):
                wait_scatter(i)
                start_gather(i, base + 4 + i)
            return carry

        lax.fori_loop(0, nch // 4 - 1, body, 0)
        base = nch - 4
        for i in range(4):
            wait_gather(i)
            start_scatter(i, base + i)
        for i in range(4):
            wait_scatter(i)
    with jax.named_scope("gs_wb"):
        plsc.subcore_barrier()
        pltpu.sync_copy(acc.at[pl.ds(lo, RPT)],
                        out.at[cid].at[pl.ds(lo, RPT)])


@functools.lru_cache(maxsize=None)
def _build_sc():
    mesh = plsc.VectorSubcoreMesh(core_axis_name="c", subcore_axis_name="s")
    params = pltpu.CompilerParams(use_tc_tiling_on_sc=False)
    sc_deg = pl.kernel(
        _sc_deg_body,
        out_type=jax.ShapeDtypeStruct((NC, N_PAD, 16), jnp.float32),
        mesh=mesh,
        compiler_params=params,
        scratch_types=[
            pltpu.VMEM((NCHUNK, C), jnp.int32),
            pltpu.VMEM((C, 16), jnp.float32),
            pltpu.VMEM_SHARED((N_PAD, 16), jnp.float32),
            pltpu.SemaphoreType.DMA,
        ],
    )
    sc_gs = pl.kernel(
        _sc_gs_body,
        out_type=jax.ShapeDtypeStruct((NC, N_PAD, H), jnp.float32),
        mesh=mesh,
        compiler_params=params,
        scratch_types=[
            pltpu.VMEM((K0, C), jnp.int32),
            pltpu.VMEM((K0, C), jnp.int32),
            pltpu.VMEM((C, H), jnp.float32),
            pltpu.VMEM((C, H), jnp.float32),
            pltpu.VMEM((C, H), jnp.float32),
            pltpu.VMEM((C, H), jnp.float32),
            pltpu.VMEM_SHARED((N_PAD, H), jnp.float32),
        ] + [pltpu.SemaphoreType.DMA] * 8,
    )
    return sc_deg, sc_gs


def _tc_prep_body(degp_ref, x_ref, w_ref, dinv_ref, g_ref):
    degp = degp_ref[...]
    deg = degp[0, :, 0:1] + degp[1, :, 0:1] + 1.0
    dinv = lax.rsqrt(jnp.maximum(deg, 1.0))
    h = jnp.dot(x_ref[...], w_ref[...], preferred_element_type=jnp.float32)
    dinv_ref[...] = dinv
    g_ref[...] = h * dinv


def _tc_mid_body(p_ref, g_ref, dinv_ref, b_ref, w_ref, gout_ref):
    p = p_ref[...]
    dinv = dinv_ref[...]
    accum = p[0] + p[1] + g_ref[...]
    xn = jnp.maximum(accum * dinv + b_ref[...], 0.0)
    gout_ref[...] = (
        jnp.dot(xn, w_ref[...], preferred_element_type=jnp.float32) * dinv
    )


def _tc_final_body(p_ref, g_ref, dinv_ref, b_ref, batch_ref, out_ref):
    p = p_ref[...]
    accum = p[0] + p[1] + g_ref[...]
    h = jnp.maximum(accum * dinv_ref[...] + b_ref[...], 0.0)
    seg = batch_ref[...]
    ids = lax.broadcasted_iota(jnp.int32, (B, N_PAD), 0)
    oh = (ids == seg).astype(jnp.float32)
    sums = jnp.dot(oh, h, preferred_element_type=jnp.float32)
    cnt = jnp.sum(oh, axis=1, keepdims=True)
    out_ref[...] = sums / jnp.maximum(cnt, 1.0)


_tc_prep = pl.pallas_call(
    _tc_prep_body,
    out_shape=[
        jax.ShapeDtypeStruct((N_PAD, 1), jnp.float32),
        jax.ShapeDtypeStruct((N_PAD, H), jnp.float32),
    ],
)

_tc_mid = pl.pallas_call(
    _tc_mid_body,
    out_shape=jax.ShapeDtypeStruct((N_PAD, H), jnp.float32),
)

_tc_final = pl.pallas_call(
    _tc_final_body,
    out_shape=jax.ShapeDtypeStruct((B, H), jnp.float32),
)


def kernel(x, edge_index, batch, W1, b1, W2, b2, W3, b3, W4, b4):
    src = edge_index[0]
    dst = edge_index[1]
    ppw = (E_PAD - E) // NW
    rpw = E // NW
    pad_src = jnp.broadcast_to(jnp.arange(ppw, dtype=jnp.int32), (NW, ppw))
    pad_dst = pad_src + N
    src2 = jnp.concatenate(
        [src.reshape(NW, rpw), pad_src], axis=1).reshape(TOT_CHUNKS, C)
    dst2 = jnp.concatenate(
        [dst.reshape(NW, rpw), pad_dst], axis=1).reshape(TOT_CHUNKS, C)
    xp = jnp.pad(x, ((0, N_PAD - N), (0, 0)))
    batch_row = jnp.pad(batch, (0, N_PAD - N),
                        constant_values=B).reshape(1, N_PAD)
    ones16 = jnp.ones((C, 16), jnp.float32)
    zeros16 = jnp.zeros((RPT, 16), jnp.float32)
    zrows = jnp.zeros((RPT, H), jnp.float32)
    b1r, b2r, b3r, b4r = (b.reshape(1, H) for b in (b1, b2, b3, b4))

    sc_deg, sc_gs = _build_sc()
    degp = sc_deg(dst2, ones16, zeros16)
    dinv, g = _tc_prep(degp, xp, W1)
    p = sc_gs(g, src2, dst2, zrows)
    g = _tc_mid(p, g, dinv, b1r, W2)
    p = sc_gs(g, src2, dst2, zrows)
    g = _tc_mid(p, g, dinv, b2r, W3)
    p = sc_gs(g, src2, dst2, zrows)
    g = _tc_mid(p, g, dinv, b3r, W4)
    p = sc_gs(g, src2, dst2, zrows)
    return _tc_final(p, g, dinv, b4r, batch_row)


# speedup vs baseline: 2.8695x; 1.2028x over previous
import functools

import jax
import jax.numpy as jnp
from jax import lax
from jax.experimental import pallas as pl
from jax.experimental.pallas import tpu as pltpu
from jax.experimental.pallas import tpu_sc as plsc

N = 10000
E = 320000
D = 128
H = 64
B = 64

NC = 2
NS = 16
NW = NC * NS
C = 128
NCHUNK = 80
E_PAD = NW * NCHUNK * C
TOT_CHUNKS = E_PAD // C
K0 = NCHUNK
K1 = 2 * NCHUNK - K0
N_PAD = 10240
RPT = N_PAD // NS


def _sc_deg_body(dst2, ones_hbm, zero16_hbm, out, didx, onesv, acc, sem):
    cid = lax.axis_index("c")
    sid = lax.axis_index("s")
    wid = cid * NS + sid
    lo = sid * RPT
    pltpu.sync_copy(zero16_hbm, acc.at[pl.ds(lo, RPT)])
    pltpu.sync_copy(ones_hbm, onesv)
    pltpu.sync_copy(dst2.at[pl.ds(wid * NCHUNK, NCHUNK)], didx)
    plsc.subcore_barrier()

    def body(bi, carry):
        descs = [
            pltpu.async_copy(onesv, acc.at[didx.at[8 * bi + i]], sem, add=True)
            for i in range(8)
        ]
        for d in descs:
            d.wait()
        return carry

    lax.fori_loop(0, NCHUNK // 8, body, 0)
    plsc.subcore_barrier()
    pltpu.sync_copy(acc.at[pl.ds(lo, RPT)], out.at[cid].at[pl.ds(lo, RPT)])


def _sc_gs_body(g, src2, dst2, zrows_hbm, out,
                sidx, didx, r0, r1, r2, r3,
                acc, g0, g1, g2, g3, s0, s1, s2, s3):
    cid = lax.axis_index("c")
    sid = lax.axis_index("s")
    lo = sid * RPT
    rows = (r0, r1, r2, r3)
    gsem = (g0, g1, g2, g3)
    ssem = (s0, s1, s2, s3)
    nch = jnp.where(cid == 0, K0, K1)
    with jax.named_scope("gs_init"):
        pltpu.async_copy(zrows_hbm, acc.at[pl.ds(lo, RPT)], s0)

        @pl.when(cid == 0)
        def _():
            pltpu.async_copy(src2.at[pl.ds(sid * K0, K0)],
                             sidx.at[pl.ds(0, K0)], s1)
            pltpu.async_copy(dst2.at[pl.ds(sid * K0, K0)],
                             didx.at[pl.ds(0, K0)], s2)

        @pl.when(cid != 0)
        def _():
            base = NS * K0 + sid * K1
            pltpu.async_copy(src2.at[pl.ds(base, K1)],
                             sidx.at[pl.ds(0, K1)], s1)
            pltpu.async_copy(dst2.at[pl.ds(base, K1)],
                             didx.at[pl.ds(0, K1)], s2)

        pltpu.make_async_copy(zrows_hbm, acc.at[pl.ds(lo, RPT)], s0).wait()
        pltpu.make_async_copy(src2.at[pl.ds(0, K0)],
                              sidx.at[pl.ds(0, K0)], s1).wait()
        pltpu.make_async_copy(dst2.at[pl.ds(0, K0)],
                              didx.at[pl.ds(0, K0)], s2).wait()
        plsc.subcore_barrier()

    def start_gather(i, chunk):
        pltpu.async_copy(g.at[sidx.at[chunk]], rows[i], gsem[i])

    def wait_gather(i):
        pltpu.make_async_copy(g.at[sidx.at[0]], rows[i], gsem[i]).wait()

    def start_scatter(i, chunk):
        pltpu.async_copy(rows[i], acc.at[didx.at[chunk]], ssem[i], add=True)

    def wait_scatter(i):
        pltpu.make_async_copy(rows[i], acc.at[didx.at[0]], ssem[i]).wait()

    with jax.named_scope("gs_loop"):
        for i in range(4):
            start_gather(i, i)

        def body(j, carry):
            base = 4 * j
            for i in range(4):
                wait_gather(i)
                start_scatter(i, base + i)
            for i in range(4):
                wait_scatter(i)
                start_gather(i, base + 4 + i)
            return carry

        lax.fori_loop(0, nch // 4 - 1, body, 0)
        base = nch - 4
        for i in range(4):
            wait_gather(i)
            start_scatter(i, base + i)
        for i in range(4):
            wait_scatter(i)
    with jax.named_scope("gs_wb"):
        plsc.subcore_barrier()
        pltpu.sync_copy(acc.at[pl.ds(lo, RPT)],
                        out.at[cid].at[pl.ds(lo, RPT)])


@functools.lru_cache(maxsize=None)
def _build_sc():
    mesh = plsc.VectorSubcoreMesh(core_axis_name="c", subcore_axis_name="s")
    params = pltpu.CompilerParams(use_tc_tiling_on_sc=False)
    sc_deg = pl.kernel(
        _sc_deg_body,
        out_type=jax.ShapeDtypeStruct((NC, N_PAD, 16), jnp.float32),
        mesh=mesh,
        compiler_params=params,
        scratch_types=[
            pltpu.VMEM((NCHUNK, C), jnp.int32),
            pltpu.VMEM((C, 16), jnp.float32),
            pltpu.VMEM_SHARED((N_PAD, 16), jnp.float32),
            pltpu.SemaphoreType.DMA,
        ],
    )
    sc_gs = pl.kernel(
        _sc_gs_body,
        out_type=jax.ShapeDtypeStruct((NC, N_PAD, H), jnp.float32),
        mesh=mesh,
        compiler_params=params,
        scratch_types=[
            pltpu.VMEM((K0, C), jnp.int32),
            pltpu.VMEM((K0, C), jnp.int32),
            pltpu.VMEM((C, H), jnp.float32),
            pltpu.VMEM((C, H), jnp.float32),
            pltpu.VMEM((C, H), jnp.float32),
            pltpu.VMEM((C, H), jnp.float32),
            pltpu.VMEM_SHARED((N_PAD, H), jnp.float32),
        ] + [pltpu.SemaphoreType.DMA] * 8,
    )
    return sc_deg, sc_gs


NP2 = N_PAD // 2


def _dinv2(de, do):
    return jnp.concatenate(
        [jnp.broadcast_to(de, (NP2, H)), jnp.broadcast_to(do, (NP2, H))],
        axis=1)


def _blockdiag(w):
    k = w.shape[0]
    z = jnp.zeros((k, H), jnp.float32)
    return jnp.concatenate(
        [jnp.concatenate([w, z], axis=1), jnp.concatenate([z, w], axis=1)],
        axis=0)


def _tc_prep_body(degp2_ref, x2_ref, w_ref, de_ref, do_ref, g_ref):
    d = degp2_ref[...]
    de = lax.rsqrt(d[0, :, 0:1] + d[1, :, 0:1] + 1.0)
    do = lax.rsqrt(d[0, :, 16:17] + d[1, :, 16:17] + 1.0)
    dinv2 = _dinv2(de, do)
    w2 = _blockdiag(w_ref[...])
    g_ref[...] = jnp.dot(x2_ref[...], w2,
                         preferred_element_type=jnp.float32) * dinv2
    de_ref[...] = de
    do_ref[...] = do


def _tc_mid_body(p2_ref, g_ref, de_ref, do_ref, b2_ref, w_ref, gout_ref):
    p = p2_ref[...]
    accum = p[0] + p[1] + g_ref[...]
    dinv2 = _dinv2(de_ref[...], do_ref[...])
    xn = jnp.maximum(accum * dinv2 + b2_ref[...], 0.0)
    w2 = _blockdiag(w_ref[...])
    gout_ref[...] = jnp.dot(xn, w2,
                            preferred_element_type=jnp.float32) * dinv2


def _tc_final_body(p2_ref, g_ref, de_ref, do_ref, b2_ref, be_ref, bo_ref,
                   out_ref):
    p = p2_ref[...]
    accum = p[0] + p[1] + g_ref[...]
    dinv2 = _dinv2(de_ref[...], do_ref[...])
    h = jnp.maximum(accum * dinv2 + b2_ref[...], 0.0)
    ids = lax.broadcasted_iota(jnp.int32, (B, NP2), 0)
    ohe = (ids == be_ref[...]).astype(jnp.float32)
    oho = (ids == bo_ref[...]).astype(jnp.float32)
    sums = (jnp.dot(ohe, h[:, :H], preferred_element_type=jnp.float32)
            + jnp.dot(oho, h[:, H:], preferred_element_type=jnp.float32))
    cnt = (jnp.sum(ohe, axis=1, keepdims=True)
           + jnp.sum(oho, axis=1, keepdims=True))
    out_ref[...] = sums / jnp.maximum(cnt, 1.0)


_tc_prep = pl.pallas_call(
    _tc_prep_body,
    out_shape=[
        jax.ShapeDtypeStruct((NP2, 1), jnp.float32),
        jax.ShapeDtypeStruct((NP2, 1), jnp.float32),
        jax.ShapeDtypeStruct((NP2, 2 * H), jnp.float32),
    ],
)

_tc_mid = pl.pallas_call(
    _tc_mid_body,
    out_shape=jax.ShapeDtypeStruct((NP2, 2 * H), jnp.float32),
)

_tc_final = pl.pallas_call(
    _tc_final_body,
    out_shape=jax.ShapeDtypeStruct((B, H), jnp.float32),
)


def kernel(x, edge_index, batch, W1, b1, W2, b2, W3, b3, W4, b4):
    src = edge_index[0]
    dst = edge_index[1]
    ppw = (E_PAD - E) // NW
    rpw = E // NW
    pad_src = jnp.broadcast_to(jnp.arange(ppw, dtype=jnp.int32), (NW, ppw))
    pad_dst = pad_src + N
    src2 = jnp.concatenate(
        [src.reshape(NW, rpw), pad_src], axis=1).reshape(TOT_CHUNKS, C)
    dst2 = jnp.concatenate(
        [dst.reshape(NW, rpw), pad_dst], axis=1).reshape(TOT_CHUNKS, C)
    x2 = jnp.pad(x, ((0, N_PAD - N), (0, 0))).reshape(NP2, 2 * D)
    batch_pad = jnp.pad(batch, (0, N_PAD - N), constant_values=B)
    be = batch_pad[0::2].reshape(1, NP2)
    bo = batch_pad[1::2].reshape(1, NP2)
    ones16 = jnp.ones((C, 16), jnp.float32)
    zeros16 = jnp.zeros((RPT, 16), jnp.float32)
    zrows = jnp.zeros((RPT, H), jnp.float32)
    b1r, b2r, b3r, b4r = (
        jnp.concatenate([b, b]).reshape(1, 2 * H) for b in (b1, b2, b3, b4))

    sc_deg, sc_gs = _build_sc()
    degp = sc_deg(dst2, ones16, zeros16)
    de, do, g = _tc_prep(degp.reshape(NC, NP2, 32), x2, W1)
    p = sc_gs(g.reshape(N_PAD, H), src2, dst2, zrows)
    g = _tc_mid(p.reshape(NC, NP2, 2 * H), g, de, do, b1r, W2)
    p = sc_gs(g.reshape(N_PAD, H), src2, dst2, zrows)
    g = _tc_mid(p.reshape(NC, NP2, 2 * H), g, de, do, b2r, W3)
    p = sc_gs(g.reshape(N_PAD, H), src2, dst2, zrows)
    g = _tc_mid(p.reshape(NC, NP2, 2 * H), g, de, do, b3r, W4)
    p = sc_gs(g.reshape(N_PAD, H), src2, dst2, zrows)
    return _tc_final(p.reshape(NC, NP2, 2 * H), g, de, do, b4r, be, bo)


# speedup vs baseline: 2.8703x; 1.0003x over previous
import functools

import jax
import jax.numpy as jnp
from jax import lax
from jax.experimental import pallas as pl
from jax.experimental.pallas import tpu as pltpu
from jax.experimental.pallas import tpu_sc as plsc

N = 10000
E = 320000
D = 128
H = 64
B = 64

NC = 2
NS = 16
NW = NC * NS
C = 128
NCHUNK = 80
E_PAD = NW * NCHUNK * C
TOT_CHUNKS = E_PAD // C
K0 = NCHUNK
K1 = 2 * NCHUNK - K0
N_PAD = 10240
RPT = N_PAD // NS


def _sc_deg_body(dst2, ones_hbm, zero16_hbm, out, didx, onesv, acc, sem):
    cid = lax.axis_index("c")
    sid = lax.axis_index("s")
    wid = cid * NS + sid
    lo = sid * RPT
    pltpu.sync_copy(zero16_hbm, acc.at[pl.ds(lo, RPT)])
    pltpu.sync_copy(ones_hbm, onesv)
    pltpu.sync_copy(dst2.at[pl.ds(wid * NCHUNK, NCHUNK)], didx)
    plsc.subcore_barrier()

    def body(bi, carry):
        descs = [
            pltpu.async_copy(onesv, acc.at[didx.at[8 * bi + i]], sem, add=True)
            for i in range(8)
        ]
        for d in descs:
            d.wait()
        return carry

    lax.fori_loop(0, NCHUNK // 8, body, 0)
    plsc.subcore_barrier()
    pltpu.sync_copy(acc.at[pl.ds(lo, RPT)], out.at[cid].at[pl.ds(lo, RPT)])


def _sc_gs_body(g, src2, dst2, zrows_hbm, out,
                sidx, didx, r0, r1, r2, r3,
                acc, g0, g1, g2, g3, s0, s1, s2, s3):
    cid = lax.axis_index("c")
    sid = lax.axis_index("s")
    lo = sid * RPT
    rows = (r0, r1, r2, r3)
    gsem = (g0, g1, g2, g3)
    ssem = (s0, s1, s2, s3)
    nch = jnp.where(cid == 0, K0, K1)
    pltpu.async_copy(zrows_hbm, acc.at[pl.ds(lo, RPT)], s0)

    @pl.when(cid == 0)
    def _():
        pltpu.async_copy(src2.at[pl.ds(sid * K0, K0)],
                         sidx.at[pl.ds(0, K0)], s1)
        pltpu.async_copy(dst2.at[pl.ds(sid * K0, K0)],
                         didx.at[pl.ds(0, K0)], s2)

    @pl.when(cid != 0)
    def _():
        base = NS * K0 + sid * K1
        pltpu.async_copy(src2.at[pl.ds(base, K1)],
                         sidx.at[pl.ds(0, K1)], s1)
        pltpu.async_copy(dst2.at[pl.ds(base, K1)],
                         didx.at[pl.ds(0, K1)], s2)

    pltpu.make_async_copy(zrows_hbm, acc.at[pl.ds(lo, RPT)], s0).wait()
    pltpu.make_async_copy(src2.at[pl.ds(0, K0)],
                          sidx.at[pl.ds(0, K0)], s1).wait()
    pltpu.make_async_copy(dst2.at[pl.ds(0, K0)],
                          didx.at[pl.ds(0, K0)], s2).wait()
    plsc.subcore_barrier()

    def start_gather(i, chunk):
        pltpu.async_copy(g.at[sidx.at[chunk]], rows[i], gsem[i])

    def wait_gather(i):
        pltpu.make_async_copy(g.at[sidx.at[0]], rows[i], gsem[i]).wait()

    def start_scatter(i, chunk):
        pltpu.async_copy(rows[i], acc.at[didx.at[chunk]], ssem[i], add=True)

    def wait_scatter(i):
        pltpu.make_async_copy(rows[i], acc.at[didx.at[0]], ssem[i]).wait()

    for i in range(4):
        start_gather(i, i)

    def body(j, carry):
        base = 4 * j
        for i in range(4):
            wait_gather(i)
            start_scatter(i, base + i)
        for i in range(4):
            wait_scatter(i)
            start_gather(i, base + 4 + i)
        return carry

    lax.fori_loop(0, nch // 4 - 1, body, 0)
    base = nch - 4
    for i in range(4):
        wait_gather(i)
        start_scatter(i, base + i)
    for i in range(4):
        wait_scatter(i)
    plsc.subcore_barrier()
    pltpu.sync_copy(acc.at[pl.ds(lo, RPT)],
                    out.at[cid].at[pl.ds(lo, RPT)])


@functools.lru_cache(maxsize=None)
def _build_sc():
    mesh = plsc.VectorSubcoreMesh(core_axis_name="c", subcore_axis_name="s")
    params = pltpu.CompilerParams(use_tc_tiling_on_sc=False)
    sc_deg = pl.kernel(
        _sc_deg_body,
        out_type=jax.ShapeDtypeStruct((NC, N_PAD, 16), jnp.float32),
        mesh=mesh,
        compiler_params=params,
        scratch_types=[
            pltpu.VMEM((NCHUNK, C), jnp.int32),
            pltpu.VMEM((C, 16), jnp.float32),
            pltpu.VMEM_SHARED((N_PAD, 16), jnp.float32),
            pltpu.SemaphoreType.DMA,
        ],
    )
    sc_gs = pl.kernel(
        _sc_gs_body,
        out_type=jax.ShapeDtypeStruct((NC, N_PAD, H), jnp.float32),
        mesh=mesh,
        compiler_params=params,
        scratch_types=[
            pltpu.VMEM((K0, C), jnp.int32),
            pltpu.VMEM((K0, C), jnp.int32),
            pltpu.VMEM((C, H), jnp.float32),
            pltpu.VMEM((C, H), jnp.float32),
            pltpu.VMEM((C, H), jnp.float32),
            pltpu.VMEM((C, H), jnp.float32),
            pltpu.VMEM_SHARED((N_PAD, H), jnp.float32),
        ] + [pltpu.SemaphoreType.DMA] * 8,
    )
    return sc_deg, sc_gs


NP2 = N_PAD // 2


def _dinv2(de, do):
    return jnp.concatenate(
        [jnp.broadcast_to(de, (NP2, H)), jnp.broadcast_to(do, (NP2, H))],
        axis=1)


def _blockdiag(w):
    k = w.shape[0]
    z = jnp.zeros((k, H), jnp.float32)
    return jnp.concatenate(
        [jnp.concatenate([w, z], axis=1), jnp.concatenate([z, w], axis=1)],
        axis=0)


def _tc_mm_body(x2_ref, w_ref, h_ref):
    w2 = _blockdiag(w_ref[...])
    h_ref[...] = jnp.dot(x2_ref[...], w2,
                         preferred_element_type=jnp.float32)


def _tc_prep_body(degp2_ref, h_ref, de_ref, do_ref, g_ref):
    d = degp2_ref[...]
    de = lax.rsqrt(d[0, :, 0:1] + d[1, :, 0:1] + 1.0)
    do = lax.rsqrt(d[0, :, 16:17] + d[1, :, 16:17] + 1.0)
    dinv2 = _dinv2(de, do)
    g_ref[...] = h_ref[...] * dinv2
    de_ref[...] = de
    do_ref[...] = do


def _tc_mid_body(p2_ref, g_ref, de_ref, do_ref, b2_ref, w_ref, gout_ref):
    p = p2_ref[...]
    accum = p[0] + p[1] + g_ref[...]
    dinv2 = _dinv2(de_ref[...], do_ref[...])
    xn = jnp.maximum(accum * dinv2 + b2_ref[...], 0.0)
    w2 = _blockdiag(w_ref[...])
    gout_ref[...] = jnp.dot(xn, w2,
                            preferred_element_type=jnp.float32) * dinv2


def _tc_final_body(p2_ref, g_ref, de_ref, do_ref, b2_ref, be_ref, bo_ref,
                   out_ref):
    p = p2_ref[...]
    accum = p[0] + p[1] + g_ref[...]
    dinv2 = _dinv2(de_ref[...], do_ref[...])
    h = jnp.maximum(accum * dinv2 + b2_ref[...], 0.0)
    ids = lax.broadcasted_iota(jnp.int32, (B, NP2), 0)
    ohe = (ids == be_ref[...]).astype(jnp.float32)
    oho = (ids == bo_ref[...]).astype(jnp.float32)
    sums = (jnp.dot(ohe, h[:, :H], preferred_element_type=jnp.float32)
            + jnp.dot(oho, h[:, H:], preferred_element_type=jnp.float32))
    cnt = (jnp.sum(ohe, axis=1, keepdims=True)
           + jnp.sum(oho, axis=1, keepdims=True))
    out_ref[...] = sums / jnp.maximum(cnt, 1.0)


_tc_mm = pl.pallas_call(
    _tc_mm_body,
    out_shape=jax.ShapeDtypeStruct((NP2, 2 * H), jnp.float32),
)

_tc_prep = pl.pallas_call(
    _tc_prep_body,
    out_shape=[
        jax.ShapeDtypeStruct((NP2, 1), jnp.float32),
        jax.ShapeDtypeStruct((NP2, 1), jnp.float32),
        jax.ShapeDtypeStruct((NP2, 2 * H), jnp.float32),
    ],
)

_tc_mid = pl.pallas_call(
    _tc_mid_body,
    out_shape=jax.ShapeDtypeStruct((NP2, 2 * H), jnp.float32),
)

_tc_final = pl.pallas_call(
    _tc_final_body,
    out_shape=jax.ShapeDtypeStruct((B, H), jnp.float32),
)


def kernel(x, edge_index, batch, W1, b1, W2, b2, W3, b3, W4, b4):
    src = edge_index[0]
    dst = edge_index[1]
    ppw = (E_PAD - E) // NW
    rpw = E // NW
    pad_src = jnp.broadcast_to(jnp.arange(ppw, dtype=jnp.int32), (NW, ppw))
    pad_dst = pad_src + N
    src2 = jnp.concatenate(
        [src.reshape(NW, rpw), pad_src], axis=1).reshape(TOT_CHUNKS, C)
    dst2 = jnp.concatenate(
        [dst.reshape(NW, rpw), pad_dst], axis=1).reshape(TOT_CHUNKS, C)
    x2 = jnp.pad(x, ((0, N_PAD - N), (0, 0))).reshape(NP2, 2 * D)
    batch_pad = jnp.pad(batch, (0, N_PAD - N), constant_values=B)
    be = batch_pad[0::2].reshape(1, NP2)
    bo = batch_pad[1::2].reshape(1, NP2)
    ones16 = jnp.ones((C, 16), jnp.float32)
    zeros16 = jnp.zeros((RPT, 16), jnp.float32)
    zrows = jnp.zeros((RPT, H), jnp.float32)
    b1r, b2r, b3r, b4r = (
        jnp.concatenate([b, b]).reshape(1, 2 * H) for b in (b1, b2, b3, b4))

    sc_deg, sc_gs = _build_sc()
    degp = sc_deg(dst2, ones16, zeros16)
    h1 = _tc_mm(x2, W1)
    de, do, g = _tc_prep(degp.reshape(NC, NP2, 32), h1)
    p = sc_gs(g.reshape(N_PAD, H), src2, dst2, zrows)
    g = _tc_mid(p.reshape(NC, NP2, 2 * H), g, de, do, b1r, W2)
    p = sc_gs(g.reshape(N_PAD, H), src2, dst2, zrows)
    g = _tc_mid(p.reshape(NC, NP2, 2 * H), g, de, do, b2r, W3)
    p = sc_gs(g.reshape(N_PAD, H), src2, dst2, zrows)
    g = _tc_mid(p.reshape(NC, NP2, 2 * H), g, de, do, b3r, W4)
    p = sc_gs(g.reshape(N_PAD, H), src2, dst2, zrows)
    return _tc_final(p.reshape(NC, NP2, 2 * H), g, de, do, b4r, be, bo)
